# Initial kernel scaffold; baseline (speedup 1.0000x reference)
import jax
import jax.numpy as jnp
from jax.experimental import pallas as pl


def kernel(x, table):
    raise NotImplementedError("write your pallas kernel here")


# speedup vs baseline: 1.3518x; 1.3518x over previous
import jax
import jax.numpy as jnp
from jax.experimental import pallas as pl
from jax.experimental.pallas import tpu as pltpu
from jax.experimental.pallas import tpu_sc as plsc

_WINDOW = 128


def _sc_gather(table, idx2d, n, d):
    mesh = plsc.VectorSubcoreMesh(
        core_axis_name="core", subcore_axis_name="subcore"
    )

    @pl.kernel(
        out_type=jax.ShapeDtypeStruct((n, d), jnp.float32),
        mesh=mesh,
        compiler_params=pltpu.CompilerParams(use_tc_tiling_on_sc=False),
    )
    def k(t_hbm, i_hbm, o_hbm):
        def body(i_vmem, o_vmem):
            pltpu.sync_copy(t_hbm.at[i_vmem.at[0]], o_vmem)

        pltpu.emit_pipeline(
            body,
            grid=(n // _WINDOW,),
            in_specs=[pl.BlockSpec((1, _WINDOW), index_map=lambda i: (0, i))],
            out_specs=[pl.BlockSpec((_WINDOW, d), index_map=lambda i: (i, 0))],
            core_axis_name=("core", "subcore"),
            dimension_semantics=(pltpu.PARALLEL,),
        )(i_hbm, o_hbm)

    return k(table, idx2d)


def kernel(x, table):
    b, s = x.shape
    v, d = table.shape
    n = b * s
    t = table.at[0].set(0.0)
    out = _sc_gather(t, x.reshape(1, n), n, d)
    return out.reshape(b, s, d)


# speedup vs baseline: 1.7893x; 1.3237x over previous
import jax
import jax.numpy as jnp
from jax import lax
from jax.experimental import pallas as pl
from jax.experimental.pallas import tpu as pltpu
from jax.experimental.pallas import tpu_sc as plsc

_NC, _NS = 2, 16
_NW = _NC * _NS
_L = 16
_CHUNK = 800


def _sc_lookup(table, idx, n, v, d):
    bpw = n // _NW
    nchunk = bpw // _CHUNK
    mesh = plsc.VectorSubcoreMesh(
        core_axis_name="core", subcore_axis_name="subcore"
    )

    @pl.kernel(
        out_type=jax.ShapeDtypeStruct((n, d), jnp.float32),
        mesh=mesh,
        compiler_params=pltpu.CompilerParams(
            use_tc_tiling_on_sc=False, needs_layout_passes=False
        ),
        scratch_types=[
            pltpu.VMEM((v, d), jnp.float32),
            pltpu.VMEM((bpw,), jnp.int32),
            pltpu.VMEM((_CHUNK, d), jnp.float32),
            pltpu.VMEM((_CHUNK, d), jnp.float32),
            pltpu.SemaphoreType.DMA,
            pltpu.SemaphoreType.DMA,
        ],
    )
    def k(t_hbm, i_hbm, o_hbm, tab_v, idx_v, rows0, rows1, sem0, sem1):
        wid = lax.axis_index("subcore") * _NC + lax.axis_index("core")
        base = wid * bpw
        pltpu.sync_copy(t_hbm, tab_v)
        pltpu.sync_copy(i_hbm.at[pl.ds(base, bpw)], idx_v)

        iota = lax.iota(jnp.int32, _L)
        rows = (rows0, rows1)
        sems = (sem0, sem1)

        def do_chunk(kc, b):
            rb, sb = rows[b], sems[b]

            @pl.when(kc >= 2)
            def _():
                pltpu.make_async_copy(rb, o_hbm.at[pl.ds(0, _CHUNK)], sb).wait()

            @pl.loop(0, _CHUNK // _L)
            def _(g):
                off = kc * _CHUNK + g * _L
                idxv = idx_v[pl.ds(off, _L)]
                rowv = iota + g * _L
                for dd in range(d):
                    cols = jnp.full((_L,), dd, jnp.int32)
                    vals = plsc.load_gather(tab_v, [idxv, cols])
                    plsc.store_scatter(rb, [rowv, cols], vals)

            pltpu.async_copy(rb, o_hbm.at[pl.ds(base + kc * _CHUNK, _CHUNK)], sb)

        @pl.loop(0, nchunk, step=2)
        def _(kc):
            do_chunk(kc, 0)
            do_chunk(kc + 1, 1)

        pltpu.make_async_copy(rows0, o_hbm.at[pl.ds(0, _CHUNK)], sem0).wait()
        pltpu.make_async_copy(rows1, o_hbm.at[pl.ds(0, _CHUNK)], sem1).wait()

    return k(table, idx)


def kernel(x, table):
    b, s = x.shape
    v, d = table.shape
    n = b * s
    t = table.at[0].set(0.0)
    out = _sc_lookup(t, x.reshape(n), n, v, d)
    return out.reshape(b, s, d)


# speedup vs baseline: 4.2596x; 2.3806x over previous
import jax
import jax.numpy as jnp
from jax import lax
from jax.experimental import pallas as pl
from jax.experimental.pallas import tpu as pltpu
from jax.experimental.pallas import tpu_sc as plsc

_NC, _NS = 2, 16
_NW = _NC * _NS
_L = 16
_CHUNK = 800


def _sc_lookup(table, idx, n, v, d):
    bpw = n // _NW
    nchunk = bpw // _CHUNK
    mesh = plsc.VectorSubcoreMesh(
        core_axis_name="core", subcore_axis_name="subcore"
    )

    @pl.kernel(
        out_type=jax.ShapeDtypeStruct((n * d,), jnp.float32),
        mesh=mesh,
        compiler_params=pltpu.CompilerParams(
            use_tc_tiling_on_sc=False, needs_layout_passes=False
        ),
        scratch_types=[
            pltpu.VMEM((v, d), jnp.float32),
            pltpu.VMEM((bpw,), jnp.int32),
            pltpu.VMEM((_CHUNK * d,), jnp.float32),
            pltpu.VMEM((_CHUNK * d,), jnp.float32),
            pltpu.SemaphoreType.DMA,
            pltpu.SemaphoreType.DMA,
        ],
    )
    def k(t_hbm, i_hbm, o_hbm, tab_v, idx_v, rows0, rows1, sem0, sem1):
        wid = lax.axis_index("subcore") * _NC + lax.axis_index("core")
        base = wid * bpw
        pltpu.sync_copy(t_hbm, tab_v)
        pltpu.sync_copy(i_hbm.at[pl.ds(base, bpw)], idx_v)

        iota = lax.iota(jnp.int32, _L)
        rows = (rows0, rows1)
        sems = (sem0, sem1)

        def do_chunk(kc, b):
            rb, sb = rows[b], sems[b]

            @pl.when(kc >= 2)
            def _():
                pltpu.make_async_copy(
                    rb, o_hbm.at[pl.ds(0, _CHUNK * d)], sb
                ).wait()

            @plsc.parallel_loop(0, _CHUNK // _L)
            def _(g):
                off = kc * _CHUNK + g * _L
                idxv = idx_v[pl.ds(off, _L)]
                rowd = (iota + g * _L) * d
                for dd in range(d):
                    cols = jnp.full((_L,), dd, jnp.int32)
                    vals = plsc.load_gather(tab_v, [idxv, cols])
                    plsc.store_scatter(rb, [rowd + dd], vals)

            pltpu.async_copy(
                rb, o_hbm.at[pl.ds((base + kc * _CHUNK) * d, _CHUNK * d)], sb
            )

        @pl.loop(0, nchunk, step=2)
        def _(kc):
            do_chunk(kc, 0)
            do_chunk(kc + 1, 1)

        pltpu.make_async_copy(rows0, o_hbm.at[pl.ds(0, _CHUNK * d)], sem0).wait()
        pltpu.make_async_copy(rows1, o_hbm.at[pl.ds(0, _CHUNK * d)], sem1).wait()

    return k(table, idx)


def kernel(x, table):
    b, s = x.shape
    v, d = table.shape
    n = b * s
    out = _sc_lookup(table, x.reshape(n), n, v, d)
    return out.reshape(b, s, d)
